# Initial kernel scaffold; baseline (speedup 1.0000x reference)
import functools

import jax
import jax.numpy as jnp
from jax import lax
from jax.experimental import pallas as pl
from jax.experimental.pallas import tpu as pltpu
from jax.experimental.pallas import tpu_sc as plsc

N = 10000
E = 320000
D = 128
L = 4

NC = 2
NS = 16
NW = NC * NS
EPW = E // NW
K = 125
NCH = EPW // K
RPT = N // NS
NZ = RPT // K

_mesh = plsc.VectorSubcoreMesh(
    core_axis_name="c", subcore_axis_name="s", num_cores=NC, num_subcores=NS
)


@functools.partial(
    pl.kernel,
    out_type=jax.ShapeDtypeStruct((NC, N, D), jnp.float32),
    mesh=_mesh,
    scratch_types=[
        pltpu.VMEM((NCH, K), jnp.int32),
        pltpu.VMEM((NCH, K), jnp.int32),
        pltpu.VMEM((K, D), jnp.float32),
        pltpu.VMEM_SHARED((N, D), jnp.float32),
        pltpu.SemaphoreType.DMA,
    ],
)
def _sc_scatter(ei_hbm, m_hbm, out_hbm, src_v, dst_v, rows_v, agg_s, sem):
    cid = lax.axis_index("c")
    sid = lax.axis_index("s")
    wid = sid * NC + cid

    pltpu.sync_copy(ei_hbm.at[0, wid], src_v)
    pltpu.sync_copy(ei_hbm.at[1, wid], dst_v)

    def _zrow(r, carry):
        for c in range(D // 16):
            rows_v[r, pl.ds(c * 16, 16)] = jnp.zeros((16,), jnp.float32)
        return carry

    lax.fori_loop(0, K, _zrow, 0)
    base = sid * RPT
    for t in range(NZ):
        pltpu.sync_copy(rows_v, agg_s.at[pl.ds(base + t * K, K)])
    plsc.subcore_barrier()

    def _chunk(j, carry):
        pltpu.async_copy(m_hbm.at[src_v.at[j]], rows_v, sem).wait()
        pltpu.sync_copy(rows_v, agg_s.at[dst_v.at[j]], add=True)
        return carry

    lax.fori_loop(0, NCH, _chunk, 0)
    plsc.subcore_barrier()

    for t in range(NZ):
        pltpu.sync_copy(agg_s.at[pl.ds(base + t * K, K)], rows_v)
        pltpu.sync_copy(rows_v, out_hbm.at[cid, pl.ds(base + t * K, K)])


_R = 1000


def _dot(a, b, dims):
    return lax.dot_general(
        a, b, (dims, ((), ())),
        precision=lax.Precision.HIGHEST,
        preferred_element_type=jnp.float32,
    )


def _pre_body(h_ref, w_ref, whh_ref, bhh_ref, m_ref, gh_ref):
    h = h_ref[...]
    m_ref[...] = _dot(h, w_ref[...], ((1,), (0,)))
    gh_ref[...] = _dot(h, whh_ref[...], ((1,), (1,))) + bhh_ref[...]


def _tc_pre(h, w, w_hh, b_hh2):
    return pl.pallas_call(
        _pre_body,
        grid=(N // _R,),
        in_specs=[
            pl.BlockSpec((_R, D), lambda i: (i, 0)),
            pl.BlockSpec((D, D), lambda i: (0, 0)),
            pl.BlockSpec((3 * D, D), lambda i: (0, 0)),
            pl.BlockSpec((1, 3 * D), lambda i: (0, 0)),
        ],
        out_specs=[
            pl.BlockSpec((_R, D), lambda i: (i, 0)),
            pl.BlockSpec((_R, 3 * D), lambda i: (i, 0)),
        ],
        out_shape=[
            jax.ShapeDtypeStruct((N, D), jnp.float32),
            jax.ShapeDtypeStruct((N, 3 * D), jnp.float32),
        ],
    )(h, w, w_hh, b_hh2)


def _gru_body(a0_ref, a1_ref, h_ref, gh_ref, wih_ref, bih_ref,
              g_ref, bt_ref, x_ref, o_ref, *, norm):
    agg = a0_ref[0] + a1_ref[0]
    gi = _dot(agg, wih_ref[...], ((1,), (1,))) + bih_ref[...]
    gh = gh_ref[...]
    h = h_ref[...]
    r = jax.nn.sigmoid(gi[:, :D] + gh[:, :D])
    z = jax.nn.sigmoid(gi[:, D:2 * D] + gh[:, D:2 * D])
    n = jnp.tanh(gi[:, 2 * D:] + r * gh[:, 2 * D:])
    hn = (1.0 - z) * n + z * h
    if norm:
        mu = jnp.mean(hn, axis=-1, keepdims=True)
        var = jnp.mean((hn - mu) ** 2, axis=-1, keepdims=True)
        hn = (hn - mu) * lax.rsqrt(var + 1e-5) * g_ref[...] + bt_ref[...]
        hn = hn + x_ref[...]
    o_ref[...] = hn


def _tc_gru(agg2, h, gh, w_ih, b_ih2, g2, bt2, x, norm):
    return pl.pallas_call(
        functools.partial(_gru_body, norm=norm),
        grid=(N // _R,),
        in_specs=[
            pl.BlockSpec((1, _R, D), lambda i: (0, i, 0)),
            pl.BlockSpec((1, _R, D), lambda i: (1, i, 0)),
            pl.BlockSpec((_R, D), lambda i: (i, 0)),
            pl.BlockSpec((_R, 3 * D), lambda i: (i, 0)),
            pl.BlockSpec((3 * D, D), lambda i: (0, 0)),
            pl.BlockSpec((1, 3 * D), lambda i: (0, 0)),
            pl.BlockSpec((1, D), lambda i: (0, 0)),
            pl.BlockSpec((1, D), lambda i: (0, 0)),
            pl.BlockSpec((_R, D), lambda i: (i, 0)),
        ],
        out_specs=pl.BlockSpec((_R, D), lambda i: (i, 0)),
        out_shape=jax.ShapeDtypeStruct((N, D), jnp.float32),
    )(agg2, agg2, h, gh, w_ih, b_ih2, g2, bt2, x)


def kernel(x, edge_index, weight, w_ih, w_hh, b_ih, b_hh, gamma, beta):
    ei = edge_index.reshape(2, NW, NCH, K)
    b_ih2 = b_ih.reshape(1, 3 * D)
    b_hh2 = b_hh.reshape(1, 3 * D)
    g2 = gamma.reshape(1, D)
    bt2 = beta.reshape(1, D)
    h = x
    for i in range(L):
        m, gh = _tc_pre(h, weight[i], w_hh, b_hh2)
        agg2 = _sc_scatter(ei, m)
        h = _tc_gru(agg2, h, gh, w_ih, b_ih2, g2, bt2, x, norm=(i == L - 1))
    return h


# speedup vs baseline: 2.8801x; 2.8801x over previous
import functools

import jax
import jax.numpy as jnp
from jax import lax
from jax.experimental import pallas as pl
from jax.experimental.pallas import tpu as pltpu
from jax.experimental.pallas import tpu_sc as plsc

N = 10000
E = 320000
D = 128
L = 4

NC = 2
NS = 16
NW = NC * NS
K = 128
NCH = 80
E2 = NW * NCH * K
N2 = 10240
RPT = N2 // NS
RC = 128
NZ = RPT // RC

_mesh = plsc.VectorSubcoreMesh(
    core_axis_name="c", subcore_axis_name="s", num_cores=NC, num_subcores=NS
)


@functools.partial(
    pl.kernel,
    out_type=jax.ShapeDtypeStruct((NC, N2, D), jnp.float32),
    mesh=_mesh,
    scratch_types=[
        pltpu.VMEM((NCH, K), jnp.int32),
        pltpu.VMEM((NCH, K), jnp.int32),
        pltpu.VMEM((K, D), jnp.float32),
        pltpu.VMEM_SHARED((N2, D), jnp.float32),
        pltpu.SemaphoreType.DMA,
    ],
)
def _sc_scatter(ei_hbm, m_hbm, out_hbm, src_v, dst_v, rows_v, agg_s, sem):
    cid = lax.axis_index("c")
    sid = lax.axis_index("s")
    wid = sid * NC + cid

    pltpu.sync_copy(ei_hbm.at[0, wid], src_v)
    pltpu.sync_copy(ei_hbm.at[1, wid], dst_v)

    def _zrow(r, carry):
        for c in range(D // 16):
            rows_v[r, pl.ds(c * 16, 16)] = jnp.zeros((16,), jnp.float32)
        return carry

    lax.fori_loop(0, K, _zrow, 0)
    base = pl.multiple_of(sid * RPT, RC)
    for t in range(NZ):
        pltpu.sync_copy(rows_v, agg_s.at[pl.ds(base + t * RC, RC)])
    plsc.subcore_barrier()

    def _chunk(j, carry):
        pltpu.async_copy(m_hbm.at[src_v.at[j]], rows_v, sem).wait()
        pltpu.sync_copy(rows_v, agg_s.at[dst_v.at[j]], add=True)
        return carry

    lax.fori_loop(0, NCH, _chunk, 0)
    plsc.subcore_barrier()

    for t in range(NZ):
        pltpu.sync_copy(agg_s.at[pl.ds(base + t * RC, RC)], rows_v)
        pltpu.sync_copy(rows_v, out_hbm.at[cid, pl.ds(base + t * RC, RC)])


_R = 1000


def _dot(a, b, dims):
    return lax.dot_general(
        a, b, (dims, ((), ())), preferred_element_type=jnp.float32
    )


def _pre_body(h_ref, w_ref, whh_ref, bhh_ref, m_ref, gh_ref):
    h = h_ref[...]
    m_ref[...] = _dot(h, w_ref[...], ((1,), (0,)))
    gh_ref[...] = _dot(h, whh_ref[...], ((1,), (1,))) + bhh_ref[...]


def _tc_pre(h, w, w_hh, b_hh2):
    return pl.pallas_call(
        _pre_body,
        grid=(N // _R,),
        in_specs=[
            pl.BlockSpec((_R, D), lambda i: (i, 0)),
            pl.BlockSpec((D, D), lambda i: (0, 0)),
            pl.BlockSpec((3 * D, D), lambda i: (0, 0)),
            pl.BlockSpec((1, 3 * D), lambda i: (0, 0)),
        ],
        out_specs=[
            pl.BlockSpec((_R, D), lambda i: (i, 0)),
            pl.BlockSpec((_R, 3 * D), lambda i: (i, 0)),
        ],
        out_shape=[
            jax.ShapeDtypeStruct((N, D), jnp.float32),
            jax.ShapeDtypeStruct((N, 3 * D), jnp.float32),
        ],
    )(h, w, w_hh, b_hh2)


def _gru_body(a0_ref, a1_ref, h_ref, gh_ref, wih_ref, bih_ref,
              g_ref, bt_ref, x_ref, o_ref, *, norm):
    agg = a0_ref[0] + a1_ref[0]
    gi = _dot(agg, wih_ref[...], ((1,), (1,))) + bih_ref[...]
    gh = gh_ref[...]
    h = h_ref[...]
    r = jax.nn.sigmoid(gi[:, :D] + gh[:, :D])
    z = jax.nn.sigmoid(gi[:, D:2 * D] + gh[:, D:2 * D])
    n = jnp.tanh(gi[:, 2 * D:] + r * gh[:, 2 * D:])
    hn = (1.0 - z) * n + z * h
    if norm:
        mu = jnp.mean(hn, axis=-1, keepdims=True)
        var = jnp.mean((hn - mu) ** 2, axis=-1, keepdims=True)
        hn = (hn - mu) * lax.rsqrt(var + 1e-5) * g_ref[...] + bt_ref[...]
        hn = hn + x_ref[...]
    o_ref[...] = hn


def _tc_gru(agg2, h, gh, w_ih, b_ih2, g2, bt2, x, norm):
    return pl.pallas_call(
        functools.partial(_gru_body, norm=norm),
        grid=(N // _R,),
        in_specs=[
            pl.BlockSpec((1, _R, D), lambda i: (0, i, 0)),
            pl.BlockSpec((1, _R, D), lambda i: (NC - 1, i, 0)),
            pl.BlockSpec((_R, D), lambda i: (i, 0)),
            pl.BlockSpec((_R, 3 * D), lambda i: (i, 0)),
            pl.BlockSpec((3 * D, D), lambda i: (0, 0)),
            pl.BlockSpec((1, 3 * D), lambda i: (0, 0)),
            pl.BlockSpec((1, D), lambda i: (0, 0)),
            pl.BlockSpec((1, D), lambda i: (0, 0)),
            pl.BlockSpec((_R, D), lambda i: (i, 0)),
        ],
        out_specs=pl.BlockSpec((_R, D), lambda i: (i, 0)),
        out_shape=jax.ShapeDtypeStruct((N, D), jnp.float32),
    )(agg2, agg2, h, gh, w_ih, b_ih2, g2, bt2, x)


def kernel(x, edge_index, weight, w_ih, w_hh, b_ih, b_hh, gamma, beta):
    pad = jnp.broadcast_to(
        jnp.array([[0], [N2 - 1]], dtype=jnp.int32), (2, E2 - E)
    )
    ei = jnp.concatenate([edge_index, pad], axis=1).reshape(2, NW, NCH, K)
    b_ih2 = b_ih.reshape(1, 3 * D)
    b_hh2 = b_hh.reshape(1, 3 * D)
    g2 = gamma.reshape(1, D)
    bt2 = beta.reshape(1, D)
    h = x
    for i in range(L):
        m, gh = _tc_pre(h, weight[i], w_hh, b_hh2)
        agg2 = _sc_scatter(ei, m)
        h = _tc_gru(agg2, h, gh, w_ih, b_ih2, g2, bt2, x, norm=(i == L - 1))
    return h


# speedup vs baseline: 3.0310x; 1.0524x over previous
import functools

import jax
import jax.numpy as jnp
from jax import lax
from jax.experimental import pallas as pl
from jax.experimental.pallas import tpu as pltpu
from jax.experimental.pallas import tpu_sc as plsc

N = 10000
E = 320000
D = 128
L = 4

NC = 2
NS = 16
NW = NC * NS
K = 128
NCH = 80
E2 = NW * NCH * K
N2 = 10240
RPT = N2 // NS
RC = 128
NZ = RPT // RC

_mesh = plsc.VectorSubcoreMesh(
    core_axis_name="c", subcore_axis_name="s", num_cores=NC, num_subcores=NS
)


@functools.partial(
    pl.kernel,
    out_type=jax.ShapeDtypeStruct((NC, N2, D), jnp.float32),
    mesh=_mesh,
    scratch_types=[
        pltpu.VMEM((NCH // 2, K), jnp.int32),
        pltpu.VMEM((NCH // 2, K), jnp.int32),
        pltpu.VMEM((K, D), jnp.float32),
        pltpu.VMEM((K, D), jnp.float32),
        pltpu.VMEM_SHARED((N2, D), jnp.float32),
        pltpu.SemaphoreType.DMA,
        pltpu.SemaphoreType.DMA,
    ],
)
def _sc_scatter(ei_hbm, m_hbm, out_hbm, src_v, dst_v, buf_a, buf_b,
                agg_s, sem_a, sem_b):
    cid = lax.axis_index("c")
    sid = lax.axis_index("s")
    wid = sid * NC + cid
    HP = NCH // 2

    def _zrow(r, carry):
        for c in range(D // 16):
            buf_a[r, pl.ds(c * 16, 16)] = jnp.zeros((16,), jnp.float32)
        return carry

    lax.fori_loop(0, K, _zrow, 0)
    base = pl.multiple_of(sid * RPT, RC)
    for t in range(NZ):
        pltpu.sync_copy(buf_a, agg_s.at[pl.ds(base + t * RC, RC)])
    plsc.subcore_barrier()

    for p in range(2):
        pltpu.sync_copy(ei_hbm.at[0, wid, pl.ds(p * HP, HP)], src_v)
        pltpu.sync_copy(ei_hbm.at[1, wid, pl.ds(p * HP, HP)], dst_v)
        pltpu.async_copy(m_hbm.at[src_v.at[0]], buf_a, sem_a)
        pltpu.async_copy(m_hbm.at[src_v.at[1]], buf_b, sem_b)

        def _pair(i, carry):
            j0 = 2 * i
            pltpu.make_async_copy(m_hbm.at[src_v.at[j0]], buf_a, sem_a).wait()
            pltpu.sync_copy(buf_a, agg_s.at[dst_v.at[j0]], add=True)
            pltpu.async_copy(m_hbm.at[src_v.at[j0 + 2]], buf_a, sem_a)
            pltpu.make_async_copy(m_hbm.at[src_v.at[j0 + 1]], buf_b, sem_b).wait()
            pltpu.sync_copy(buf_b, agg_s.at[dst_v.at[j0 + 1]], add=True)
            pltpu.async_copy(m_hbm.at[src_v.at[j0 + 3]], buf_b, sem_b)
            return carry

        lax.fori_loop(0, HP // 2 - 1, _pair, 0)
        pltpu.make_async_copy(m_hbm.at[src_v.at[HP - 2]], buf_a, sem_a).wait()
        pltpu.sync_copy(buf_a, agg_s.at[dst_v.at[HP - 2]], add=True)
        pltpu.make_async_copy(m_hbm.at[src_v.at[HP - 1]], buf_b, sem_b).wait()
        pltpu.sync_copy(buf_b, agg_s.at[dst_v.at[HP - 1]], add=True)
    plsc.subcore_barrier()

    for t in range(NZ):
        pltpu.sync_copy(agg_s.at[pl.ds(base + t * RC, RC)], buf_a)
        pltpu.sync_copy(buf_a, out_hbm.at[cid, pl.ds(base + t * RC, RC)])


_R = 1000


def _dot(a, b, dims):
    return lax.dot_general(
        a, b, (dims, ((), ())), preferred_element_type=jnp.float32
    )


def _pre_body(h_ref, w_ref, whh_ref, bhh_ref, m_ref, gh_ref):
    h = h_ref[...]
    m_ref[...] = _dot(h, w_ref[...], ((1,), (0,)))
    gh_ref[...] = _dot(h, whh_ref[...], ((1,), (1,))) + bhh_ref[...]


def _tc_pre(h, w, w_hh, b_hh2):
    return pl.pallas_call(
        _pre_body,
        grid=(N // _R,),
        in_specs=[
            pl.BlockSpec((_R, D), lambda i: (i, 0)),
            pl.BlockSpec((D, D), lambda i: (0, 0)),
            pl.BlockSpec((3 * D, D), lambda i: (0, 0)),
            pl.BlockSpec((1, 3 * D), lambda i: (0, 0)),
        ],
        out_specs=[
            pl.BlockSpec((_R, D), lambda i: (i, 0)),
            pl.BlockSpec((_R, 3 * D), lambda i: (i, 0)),
        ],
        out_shape=[
            jax.ShapeDtypeStruct((N, D), jnp.float32),
            jax.ShapeDtypeStruct((N, 3 * D), jnp.float32),
        ],
    )(h, w, w_hh, b_hh2)


def _gru_body(a0_ref, a1_ref, h_ref, gh_ref, wih_ref, bih_ref,
              g_ref, bt_ref, x_ref, o_ref, *, norm):
    agg = a0_ref[0] + a1_ref[0]
    gi = _dot(agg, wih_ref[...], ((1,), (1,))) + bih_ref[...]
    gh = gh_ref[...]
    h = h_ref[...]
    r = jax.nn.sigmoid(gi[:, :D] + gh[:, :D])
    z = jax.nn.sigmoid(gi[:, D:2 * D] + gh[:, D:2 * D])
    n = jnp.tanh(gi[:, 2 * D:] + r * gh[:, 2 * D:])
    hn = (1.0 - z) * n + z * h
    if norm:
        mu = jnp.mean(hn, axis=-1, keepdims=True)
        var = jnp.mean((hn - mu) ** 2, axis=-1, keepdims=True)
        hn = (hn - mu) * lax.rsqrt(var + 1e-5) * g_ref[...] + bt_ref[...]
        hn = hn + x_ref[...]
    o_ref[...] = hn


def _tc_gru(agg2, h, gh, w_ih, b_ih2, g2, bt2, x, norm):
    return pl.pallas_call(
        functools.partial(_gru_body, norm=norm),
        grid=(N // _R,),
        in_specs=[
            pl.BlockSpec((1, _R, D), lambda i: (0, i, 0)),
            pl.BlockSpec((1, _R, D), lambda i: (NC - 1, i, 0)),
            pl.BlockSpec((_R, D), lambda i: (i, 0)),
            pl.BlockSpec((_R, 3 * D), lambda i: (i, 0)),
            pl.BlockSpec((3 * D, D), lambda i: (0, 0)),
            pl.BlockSpec((1, 3 * D), lambda i: (0, 0)),
            pl.BlockSpec((1, D), lambda i: (0, 0)),
            pl.BlockSpec((1, D), lambda i: (0, 0)),
            pl.BlockSpec((_R, D), lambda i: (i, 0)),
        ],
        out_specs=pl.BlockSpec((_R, D), lambda i: (i, 0)),
        out_shape=jax.ShapeDtypeStruct((N, D), jnp.float32),
    )(agg2, agg2, h, gh, w_ih, b_ih2, g2, bt2, x)


def kernel(x, edge_index, weight, w_ih, w_hh, b_ih, b_hh, gamma, beta):
    pad = jnp.broadcast_to(
        jnp.array([[0], [N2 - 1]], dtype=jnp.int32), (2, E2 - E)
    )
    ei = jnp.concatenate([edge_index, pad], axis=1).reshape(2, NW, NCH, K)
    b_ih2 = b_ih.reshape(1, 3 * D)
    b_hh2 = b_hh.reshape(1, 3 * D)
    g2 = gamma.reshape(1, D)
    bt2 = beta.reshape(1, D)
    h = x
    for i in range(L):
        m, gh = _tc_pre(h, weight[i], w_hh, b_hh2)
        agg2 = _sc_scatter(ei, m)
        h = _tc_gru(agg2, h, gh, w_ih, b_ih2, g2, bt2, x, norm=(i == L - 1))
    return h
